# Initial kernel scaffold; baseline (speedup 1.0000x reference)
import jax
import jax.numpy as jnp
from jax.experimental import pallas as pl


def kernel(feat, label, pred, is_labelled):
    raise NotImplementedError("write your pallas kernel here")


# speedup vs baseline: 1.0224x; 1.0224x over previous
import functools

import jax
import jax.numpy as jnp
from jax import lax
from jax.experimental import pallas as pl
from jax.experimental.pallas import tpu as pltpu
from jax.experimental.pallas import tpu_sc as plsc

NUM_CLASSES = 20
NCH = 21
FEAT = 256
NPIX = 8 * 128 * 128
NW = 16
PXW = NPIX // NW
THRESH = float(0.5 + min(1.0, 1.0 / 5000.0) * (0.85 - 0.5))
LANE_STRIDE = NCH * FEAT
NEG_INF = float("-inf")
BIGI = 2 ** 30

f32 = jnp.float32
i32 = jnp.int32


def _body(f0, f1, f2, f3, f4, f5, f6, f7, predr, labelr, labr, out_hbm,
          sh_cnt, sh_top, sh_wcnt, sh_sums, sh_s2, sh_proto, sh_pn, sh_inter,
          pred_buf, lbl_buf, lab_buf, pk_arr, bank, ccnt, tcnt, wcnt,
          feat_buf, vec_buf, row_buf, row_buf2, rowj_buf, cnt_row, cb_buf,
          in_buf, out_buf, gcnt, needv, chosen_buf,
          sem0, sem1):
  core = lax.axis_index("c")
  wid = lax.axis_index("s")
  lane = lax.broadcasted_iota(i32, (16,), 0)
  lane0 = lane == 0
  lane1 = lane == 1
  zf = jnp.zeros((16,), f32)
  zi = jnp.zeros((16,), i32)
  onesf = jnp.full((16,), 1.0, f32)

  featparts = (f0, f1, f2, f3, f4, f5, f6, f7)

  def splat_f(ref, idx):
    return plsc.load_gather(ref, [zi + idx])

  @pl.when(core == 0)
  def _main():
    colbase = wid * 1024

    def _zero_block(ref, nwords):
      def zb(i, _):
        ref[pl.ds(i * 16, 16)] = zf
        return 0
      lax.fori_loop(0, nwords // 16, zb, 0)

    _zero_block(ccnt, 336)
    _zero_block(tcnt, 336)

    for bb in range(8):
      def stage_a_chunk(pc, _, bb=bb):
        sync = pltpu.sync_copy
        sync(predr.at[bb, :, pl.ds(colbase + pc * 256, 256)], pred_buf)
        goff = bb * 16384 + colbase + pc * 256
        sync(labelr.at[pl.ds(goff, 256)], lbl_buf)
        sync(labr.at[pl.ds(goff, 256)], lab_buf)

        def group(g, _):
          off = g * 16
          m = pred_buf[0, pl.ds(off, 16)]
          ci = zi
          for c in range(1, NCH):
            v = pred_buf[c, pl.ds(off, 16)]
            gt = v > m
            m = jnp.where(gt, v, m)
            ci = jnp.where(gt, c, ci)
          lbl16 = lbl_buf[pl.ds(off, 16)]
          lab16 = lab_buf[pl.ds(off, 16)]
          maskb = (m > THRESH) & (ci > 0) & ((ci == lbl16) | (lab16 == 0))
          pk = ci + jnp.where(maskb, 32, 0)
          pk_arr[pl.ds(bb * 1024 + pc * 256 + off, 16)] = pk
          w16 = jnp.where(maskb, 1.0, 0.0).astype(f32)
          idxc = ci * 16 + lane
          plsc.addupdate_scatter(ccnt, [idxc], w16)
          plsc.addupdate_scatter(tcnt, [idxc], onesf)
          return 0
        lax.fori_loop(0, 16, group, 0)
        return 0
      lax.fori_loop(0, 4, stage_a_chunk, 0)

    def _count_row(src_ref):
      cnt_row[0, pl.ds(0, 16)] = zf
      cnt_row[0, pl.ds(16, 16)] = zf

      def one_class(c, _):
        val = jnp.sum(src_ref[pl.ds(c * 16, 16)])
        plsc.store_scatter(cnt_row, [zi, zi + c], zf + val, mask=lane0)
        return 0
      lax.fori_loop(0, NCH, one_class, 0)

    _count_row(ccnt)
    pltpu.sync_copy(cnt_row, sh_cnt.at[wid, 0])
    _count_row(tcnt)
    pltpu.sync_copy(cnt_row, sh_cnt.at[wid, 1])
    plsc.subcore_barrier()

    def rc(w2, acc):
      a0, a1, a2, a3 = acc
      pltpu.sync_copy(sh_cnt.at[w2], cb_buf)
      return (a0 + cb_buf[0, 0, pl.ds(0, 16)],
              a1 + cb_buf[0, 0, pl.ds(16, 16)],
              a2 + cb_buf[1, 0, pl.ds(0, 16)],
              a3 + cb_buf[1, 0, pl.ds(16, 16)])
    acl, ach, atl, ath = lax.fori_loop(0, NW, rc, (zf, zf, zf, zf))

    needlo = jnp.where((acl < 10.0) & (atl >= 10.0) & (lane >= 1), 1.0, 0.0)
    needhi = jnp.where((ach < 10.0) & (ath >= 10.0) & (lane <= 4), 1.0, 0.0)
    needv[pl.ds(0, 16)] = needlo.astype(f32)
    needv[pl.ds(16, 16)] = needhi.astype(f32)
    any_need = (jnp.sum(needlo) + jnp.sum(needhi)) > 0.0

    @pl.when(any_need)
    def _topup():
      def per_class(c, _):
        nv = splat_f(needv, c)
        is_need = jnp.sum(jnp.where(lane0, nv, 0.0)) > 0.0

        @pl.when(is_need)
        def _one_class():
          chosen_buf[pl.ds(0, 16)] = zi - 1

          def one_round(r, _r):
            ch = [plsc.load_gather(chosen_buf, [zi + k]) for k in range(10)]

            mvmi = (jnp.full((16,), NEG_INF, f32), zi)
            for bb in range(8):
              def scan_chunk(pc, carry, bb=bb):
                mv, mi = carry
                pltpu.sync_copy(
                    predr.at[bb, :, pl.ds(colbase + pc * 256, 256)],
                    pred_buf)

                def group(g, carry2):
                  mv2, mi2 = carry2
                  off = g * 16
                  m = pred_buf[0, pl.ds(off, 16)]
                  for cc in range(1, NCH):
                    m = jnp.maximum(m, pred_buf[cc, pl.ds(off, 16)])
                  pkv = pk_arr[pl.ds(bb * 1024 + pc * 256 + off, 16)]
                  cls = jnp.where(pkv >= 32, pkv - 32, pkv)
                  gidx = bb * 16384 + colbase + pc * 256 + off + lane
                  elig = cls == c
                  for k in range(10):
                    elig = elig & (gidx != ch[k])
                  mm = jnp.where(elig, m, NEG_INF)
                  gt = mm > mv2
                  return (jnp.where(gt, mm, mv2), jnp.where(gt, gidx, mi2))
                return lax.fori_loop(0, 16, group, (mv, mi))
              mvmi = lax.fori_loop(0, 4, scan_chunk, mvmi)
            mval16, midx16 = mvmi
            mloc = jnp.max(mval16)
            iloc = jnp.min(jnp.where(mval16 == mloc, midx16, BIGI))
            vec_buf[0, pl.ds(0, 16)] = jnp.where(
                lane0, mloc, jnp.where(lane1, iloc.astype(f32), 0.0))
            pltpu.sync_copy(vec_buf, sh_top.at[wid])
            plsc.subcore_barrier()

            def pick(w2, carry):
              bm, bi = carry
              pltpu.sync_copy(sh_top.at[w2], in_buf)
              row = in_buf[0, pl.ds(0, 16)]
              mw = jnp.sum(jnp.where(lane0, row, 0.0))
              iw = jnp.sum(jnp.where(lane1, row, 0.0))
              take = (mw > bm) | ((mw == bm) & (iw < bi))
              return (jnp.where(take, mw, bm), jnp.where(take, iw, bi))
            bm, bi = lax.fori_loop(
                0, NW, pick, (jnp.float32(NEG_INF), jnp.float32(BIGI)))
            bii = bi.astype(i32)
            plsc.store_scatter(chosen_buf, [zi + r], zi + bii, mask=lane0)
            bbg = lax.shift_right_logical(bii, 14)
            col = bii & 16383
            local = bbg * 1024 + (col - colbase)

            @pl.when((col >= colbase) & (col < colbase + 1024))
            def _apply():
              plsc.store_scatter(pk_arr, [zi + local],
                                 zi + (c + 32), mask=lane0)
            plsc.subcore_barrier()
            return 0
          lax.fori_loop(0, 10, one_round, 0)
        return 0
      lax.fori_loop(1, NCH, per_class, 0)

    _zero_block(wcnt, 336)

    def wc_group(g, _):
      pkv = pk_arr[pl.ds(g * 16, 16)]
      maskb = pkv >= 32
      mcls = jnp.where(maskb, pkv - 32, 0)
      plsc.addupdate_scatter(wcnt, [mcls * 16 + lane], onesf, mask=maskb)
      return 0
    lax.fori_loop(0, PXW // 16, wc_group, 0)
    _count_row(wcnt)
    pltpu.sync_copy(cnt_row, sh_wcnt.at[wid])
    plsc.subcore_barrier()

    def rn(w2, acc):
      a0, a1 = acc
      pltpu.sync_copy(sh_wcnt.at[w2], cb_buf.at[0])
      return (a0 + cb_buf[0, 0, pl.ds(0, 16)], a1 + cb_buf[0, 0, pl.ds(16, 16)])
    nlo, nhi = lax.fori_loop(0, NW, rn, (zf, zf))
    gcnt[pl.ds(0, 16)] = nlo
    gcnt[pl.ds(16, 16)] = nhi
    denom = (jnp.sum(jnp.where(lane >= 1, nlo, 0.0))
             + jnp.sum(jnp.where(lane <= 4, nhi, 0.0)))

    _zero_block(bank, 16 * LANE_STRIDE)
    lane_off = lane * LANE_STRIDE

    s2acc = zf
    for bb in range(8):
      fp = featparts[bb]

      def chunk_loop(cb, s2c, fp=fp, bb=bb):
        for ch in range(8):
          off = (cb * 8 + ch) * 16384 + colbase
          pltpu.sync_copy(fp.at[pl.ds(off, 1024)], feat_buf.at[ch])
        cbase = cb * 8

        def group(g, s2g):
          pkv = pk_arr[pl.ds(bb * 1024 + g * 16, 16)]
          maskb = pkv >= 32
          mcls = jnp.where(maskb, pkv - 32, 0)
          w16 = jnp.where(maskb, 1.0, 0.0).astype(f32)
          base16 = lane_off + mcls * 256 + cbase
          for ch in range(8):
            f = feat_buf[ch, pl.ds(g * 16, 16)]
            fw = f * w16
            plsc.addupdate_scatter(bank, [base16 + ch], fw, mask=maskb)
            s2g = s2g + fw * f
          return s2g
        return lax.fori_loop(0, 64, group, s2c)
      s2acc = lax.fori_loop(0, 32, chunk_loop, s2acc)
    vec_buf[0, pl.ds(0, 16)] = s2acc
    pltpu.sync_copy(vec_buf, sh_s2.at[wid])

    def merge_class(c, _):
      def merge_cg(cg, _2):
        base = c * 256 + cg * 16
        acc = bank[pl.ds(base, 16)]
        for l in range(1, 16):
          acc = acc + bank[pl.ds(l * LANE_STRIDE + base, 16)]
        row_buf[0, pl.ds(cg * 16, 16)] = acc
        return 0
      lax.fori_loop(0, 16, merge_cg, 0)
      pltpu.sync_copy(row_buf, sh_sums.at[wid, c])
      return 0
    lax.fori_loop(1, NCH, merge_class, 0)
    plsc.subcore_barrier()

    def _reduce_class(c):
      def zrow(cg, _):
        row_buf2[0, pl.ds(cg * 16, 16)] = zf
        return 0
      lax.fori_loop(0, 16, zrow, 0)

      def rsum(w2, _):
        pltpu.sync_copy(sh_sums.at[w2, c], rowj_buf)

        def addcg(cg, _2):
          row_buf2[0, pl.ds(cg * 16, 16)] = (
              row_buf2[0, pl.ds(cg * 16, 16)]
              + rowj_buf[0, pl.ds(cg * 16, 16)])
          return 0
        lax.fori_loop(0, 16, addcg, 0)
        return 0
      lax.fori_loop(0, NW, rsum, 0)
      n_spl = splat_f(gcnt, c)
      rinv = (zf + 1.0) / jnp.maximum(n_spl, 1.0)

      def pncg(cg, pnv):
        p16 = row_buf2[0, pl.ds(cg * 16, 16)] * rinv
        row_buf[0, pl.ds(cg * 16, 16)] = p16
        return pnv + p16 * p16
      pnv = lax.fori_loop(0, 16, pncg, zf)
      pn_v = (zf + jnp.sum(pnv)) * n_spl
      pltpu.sync_copy(row_buf, sh_proto.at[c - 1])
      vec_buf[0, pl.ds(0, 16)] = jnp.where(lane0, pn_v, 0.0)
      pltpu.sync_copy(vec_buf, sh_pn.at[c])

    _reduce_class(wid + 1)

    @pl.when(wid < 4)
    def _second_class():
      _reduce_class(wid + 17)
    plsc.subcore_barrier()

    def _row_hinge(i):
      pltpu.sync_copy(sh_proto.at[i], row_buf)
      ni = splat_f(gcnt, i + 1)
      iv = zi + i

      def pair(j, acc):
        pltpu.sync_copy(sh_proto.at[j], rowj_buf)

        def d2cg(cg, a16):
          d = (row_buf[0, pl.ds(cg * 16, 16)]
               - rowj_buf[0, pl.ds(cg * 16, 16)])
          return a16 + d * d
        a16 = lax.fori_loop(0, 16, d2cg, zf)
        d2v = zf + (jnp.sum(a16) + 1e-8)
        yi = lax.bitcast_convert_type(d2v, i32)
        yi = 0x5F3759DF - lax.shift_right_logical(yi, 1)
        y = lax.bitcast_convert_type(yi, f32)
        for _ in range(3):
          y = y * (1.5 - 0.5 * d2v * y * y)
        distv = d2v * y
        hv = jnp.maximum(1.0 - distv, 0.0)
        hv = hv * hv
        nj = splat_f(gcnt, j + 1)
        pmv = jnp.where(((zi + j) != iv) & (ni > 0.0) & (nj > 0.0), 1.0, 0.0)
        return acc + jnp.where(lane0, hv * pmv,
                               jnp.where(lane1, pmv, 0.0))
      acc = lax.fori_loop(0, NUM_CLASSES, pair, zf)
      vec_buf[0, pl.ds(0, 16)] = acc
      pltpu.sync_copy(vec_buf, sh_inter.at[i])

    _row_hinge(wid)

    @pl.when(wid < 4)
    def _second_row():
      _row_hinge(wid + 16)
    plsc.subcore_barrier()

    @pl.when(wid == 0)
    def _final():
      def rs2(w2, s2):
        pltpu.sync_copy(sh_s2.at[w2], in_buf)
        return s2 + jnp.sum(in_buf[0, pl.ds(0, 16)])
      s2 = lax.fori_loop(0, NW, rs2, jnp.float32(0.0))

      def rpn(c, pn):
        pltpu.sync_copy(sh_pn.at[c], in_buf)
        return pn + jnp.sum(jnp.where(lane0, in_buf[0, pl.ds(0, 16)], 0.0))
      pn_tot = lax.fori_loop(1, NCH, rpn, jnp.float32(0.0))

      def rint(i, hp):
        h, p = hp
        pltpu.sync_copy(sh_inter.at[i], in_buf)
        v = in_buf[0, pl.ds(0, 16)]
        return (h + jnp.sum(jnp.where(lane0, v, 0.0)),
                p + jnp.sum(jnp.where(lane1, v, 0.0)))
      h_tot, pm_tot = lax.fori_loop(
          0, NUM_CLASSES, rint, (jnp.float32(0.0), jnp.float32(0.0)))

      intra = (zf + (s2 - pn_tot)) / jnp.maximum(zf + denom, 1.0)
      inter = (zf + h_tot) / jnp.maximum(zf + pm_tot, 1.0)
      outv = jnp.where(lane0, intra,
                       jnp.where(lane1, inter,
                                 jnp.where(lane == 2, intra + 0.1 * inter,
                                           0.0)))
      out_buf[...] = outv
      pltpu.sync_copy(out_buf, out_hbm)


_sc_call = functools.partial(
    pl.kernel,
    out_type=[
        jax.ShapeDtypeStruct((16,), f32),
        jax.ShapeDtypeStruct((NW, 2, 1, 32), f32),
        jax.ShapeDtypeStruct((NW, 1, 16), f32),
        jax.ShapeDtypeStruct((NW, 1, 32), f32),
        jax.ShapeDtypeStruct((NW, NCH, 1, 256), f32),
        jax.ShapeDtypeStruct((NW, 1, 16), f32),
        jax.ShapeDtypeStruct((NUM_CLASSES, 1, 256), f32),
        jax.ShapeDtypeStruct((NCH, 1, 16), f32),
        jax.ShapeDtypeStruct((NUM_CLASSES, 1, 16), f32),
    ],
    mesh=plsc.VectorSubcoreMesh(core_axis_name="c", subcore_axis_name="s",
                                num_cores=2, num_subcores=16),
    compiler_params=pltpu.CompilerParams(needs_layout_passes=False),
    scratch_types=[
        pltpu.VMEM((NCH, 256), f32),
        pltpu.VMEM((256,), i32),
        pltpu.VMEM((256,), i32),
        pltpu.VMEM((PXW,), i32),
        pltpu.VMEM((16 * LANE_STRIDE,), f32),
        pltpu.VMEM((336,), f32),
        pltpu.VMEM((336,), f32),
        pltpu.VMEM((336,), f32),
        pltpu.VMEM((8, 1024), f32),
        pltpu.VMEM((1, 16), f32),
        pltpu.VMEM((1, 256), f32),
        pltpu.VMEM((1, 256), f32),
        pltpu.VMEM((1, 256), f32),
        pltpu.VMEM((1, 32), f32),
        pltpu.VMEM((2, 1, 32), f32),
        pltpu.VMEM((1, 16), f32),
        pltpu.VMEM((16,), f32),
        pltpu.VMEM((32,), f32),
        pltpu.VMEM((32,), f32),
        pltpu.VMEM((16,), i32),
        pltpu.SemaphoreType.DMA,
        pltpu.SemaphoreType.DMA,
    ],
)


@jax.jit
def _run(featr, predr, labelr, labr):
  parts = [featr[k].reshape(-1) for k in range(8)]
  return _sc_call(_body)(*parts, predr, labelr, labr)[0]


def kernel(feat, label, pred, is_labelled):
  featr = feat.reshape(8, FEAT, 128 * 128)
  predr = pred.reshape(8, NCH, 128 * 128)
  labelr = label.reshape(-1)
  labr = is_labelled.reshape(-1).astype(i32)
  out16 = _run(featr, predr, labelr, labr)
  return out16[:3]
